# baseline (speedup 1.0000x reference)
import jax
import jax.numpy as jnp
from jax import lax
from jax.experimental import pallas as pl
from jax.experimental.pallas import tpu as pltpu, tpu_sc as plsc

NUM_DOMAINS = 5
EMBED_DIM = 16
BATCH = 16384
L = 16

NC = 2
NS = 16
NW = NC * NS
B_PER_W = BATCH // NW
N_BLOCKS = B_PER_W // L

_mesh = plsc.VectorSubcoreMesh(core_axis_name="c", subcore_axis_name="s")


def _body(ids_hbm, table_hbm, out_hbm, idx_v, tab_v, rows_v, sem):
    wid = lax.axis_index("s") * NC + lax.axis_index("c")
    base = wid * B_PER_W
    pltpu.sync_copy(table_hbm, tab_v)
    pltpu.sync_copy(ids_hbm.at[pl.ds(base, B_PER_W)], idx_v)
    iota = lax.iota(jnp.int32, L)

    def block(b, _):
        v_ids = idx_v[pl.ds(b * L, L)]
        v_rows = iota + b * L
        for j in range(EMBED_DIM):
            col = jnp.full((L,), j, jnp.int32)
            vals = plsc.load_gather(tab_v, [v_ids, col])
            plsc.store_scatter(rows_v, [v_rows, col], vals)
        return 0

    lax.fori_loop(0, N_BLOCKS, block, 0)
    pltpu.sync_copy(rows_v, out_hbm.at[pl.ds(base, B_PER_W)])


_sc_lookup = pl.kernel(
    _body,
    out_type=jax.ShapeDtypeStruct((BATCH, EMBED_DIM), jnp.float32),
    mesh=_mesh,
    scratch_types=[
        pltpu.VMEM((B_PER_W,), jnp.int32),
        pltpu.VMEM((NUM_DOMAINS, EMBED_DIM), jnp.float32),
        pltpu.VMEM((B_PER_W, EMBED_DIM), jnp.float32),
        pltpu.SemaphoreType.DMA,
    ],
    compiler_params=pltpu.CompilerParams(
        use_tc_tiling_on_sc=True, needs_layout_passes=False
    ),
)


@jax.jit
def kernel(domain_ids, table):
    return _sc_lookup(domain_ids.astype(jnp.int32), table)


# speedup vs baseline: 1.0349x; 1.0349x over previous
import jax
import jax.numpy as jnp
from jax import lax
from jax.experimental import pallas as pl
from jax.experimental.pallas import tpu as pltpu, tpu_sc as plsc

NUM_DOMAINS = 5
EMBED_DIM = 16
BATCH = 16384
L = 16

NC = 2
NS = 16
NW = NC * NS
B_PER_W = BATCH // NW
N_BLOCKS = B_PER_W // L

_mesh = plsc.VectorSubcoreMesh(core_axis_name="c", subcore_axis_name="s")


N_CHUNKS = 4
BLK_PER_CHUNK = N_BLOCKS // N_CHUNKS
ROWS_PER_CHUNK = BLK_PER_CHUNK * L


def _body(ids_hbm, table_hbm, out_hbm, idx_v, tab_v, rows_v, sem):
    wid = lax.axis_index("s") * NC + lax.axis_index("c")
    base = wid * B_PER_W
    c_tab = pltpu.async_copy(table_hbm, tab_v, sem)
    c_ids = pltpu.async_copy(ids_hbm.at[pl.ds(base, B_PER_W)], idx_v, sem)
    c_tab.wait()
    c_ids.wait()
    iota = lax.iota(jnp.int32, L)

    def block(b, _):
        v_ids = idx_v[pl.ds(b * L, L)]
        v_rows = iota + b * L
        for j in range(EMBED_DIM):
            col = jnp.full((L,), j, jnp.int32)
            vals = plsc.load_gather(tab_v, [v_ids, col])
            plsc.store_scatter(rows_v, [v_rows, col], vals)
        return 0

    pend = []
    for c in range(N_CHUNKS):
        lax.fori_loop(c * BLK_PER_CHUNK, (c + 1) * BLK_PER_CHUNK, block, 0)
        r0 = c * ROWS_PER_CHUNK
        pend.append(
            pltpu.async_copy(
                rows_v.at[pl.ds(r0, ROWS_PER_CHUNK)],
                out_hbm.at[pl.ds(base + r0, ROWS_PER_CHUNK)],
                sem,
            )
        )
    for p in pend:
        p.wait()


_sc_lookup = pl.kernel(
    _body,
    out_type=jax.ShapeDtypeStruct((BATCH, EMBED_DIM), jnp.float32),
    mesh=_mesh,
    scratch_types=[
        pltpu.VMEM((B_PER_W,), jnp.int32),
        pltpu.VMEM((NUM_DOMAINS, EMBED_DIM), jnp.float32),
        pltpu.VMEM((B_PER_W, EMBED_DIM), jnp.float32),
        pltpu.SemaphoreType.DMA,
    ],
    compiler_params=pltpu.CompilerParams(
        use_tc_tiling_on_sc=True, needs_layout_passes=False
    ),
)


@jax.jit
def kernel(domain_ids, table):
    return _sc_lookup(domain_ids.astype(jnp.int32), table)
